# Initial kernel scaffold; baseline (speedup 1.0000x reference)
import functools

import jax
import jax.numpy as jnp
from jax import lax
from jax.experimental import pallas as pl
from jax.experimental.pallas import tpu as pltpu
from jax.experimental.pallas import tpu_sc as plsc

N = 10000
E = 320000
D = 128
L = 4
FA = 9
FB = 3
VA = 100
VB = 5
OUT = 128
EPS = 1e-5

NC = 2
NS = 16
NT = NC * NS
EC = E // NT
BLK = 80
NBLK = EC // BLK
NROW = N // NS


def _deg_body(src2_hbm, deg2_hbm, srcf_v, ones_v, zb_v, deg_sp, sem):
    del sem
    cid = lax.axis_index("c")
    sid = lax.axis_index("s")
    chunk = cid * NS + sid

    def fill(i, _):
        ones_v[pl.ds(i * 16, 16)] = jnp.full((16,), 1.0, jnp.float32)
        return 0
    lax.fori_loop(0, EC // 16, fill, 0)

    def zfill(i, _):
        zb_v[pl.ds(i * 16, 16)] = jnp.zeros((16,), jnp.float32)
        return 0
    lax.fori_loop(0, 2000 // 16, zfill, 0)

    @pl.when(sid < 5)
    def _():
        pltpu.sync_copy(zb_v, deg_sp.at[pl.ds(sid * 2000, 2000)])

    pltpu.sync_copy(src2_hbm.at[chunk], srcf_v)
    plsc.subcore_barrier()
    pltpu.sync_copy(ones_v, deg_sp.at[srcf_v], add=True)
    plsc.subcore_barrier()

    @pl.when(sid < 5)
    def _():
        pltpu.sync_copy(deg_sp.at[pl.ds(sid * 2000, 2000)],
                        deg2_hbm.at[cid, pl.ds(sid * 2000, 2000)])


_deg_call = pl.kernel(
    _deg_body,
    out_type=jax.ShapeDtypeStruct((NC, N), jnp.float32),
    mesh=plsc.VectorSubcoreMesh(core_axis_name="c", subcore_axis_name="s"),
    scratch_types=[
        pltpu.VMEM((EC,), jnp.int32),
        pltpu.VMEM((EC,), jnp.float32),
        pltpu.VMEM((2000,), jnp.float32),
        pltpu.VMEM_SHARED((N,), jnp.float32),
        pltpu.SemaphoreType.DMA,
    ],
)


def _edge_body(xl_hbm, src3_hbm, dst3_hbm, bidx3_hbm, norm_hbm, ctab_hbm,
               out_hbm, srcv, dstv, bidxv, normv, ctabv, xrows, narr, zb,
               acc_sp, sem):
    cid = lax.axis_index("c")
    sid = lax.axis_index("s")
    chunk = cid * NS + sid
    lane = lax.iota(jnp.int32, 16)
    z16 = jnp.zeros((16,), jnp.float32)

    pltpu.sync_copy(src3_hbm.at[chunk], srcv)
    pltpu.sync_copy(dst3_hbm.at[chunk], dstv)
    pltpu.sync_copy(bidx3_hbm.at[chunk], bidxv)
    pltpu.sync_copy(norm_hbm, normv)
    pltpu.sync_copy(ctab_hbm, ctabv)

    def zfill(i, _):
        isp = jnp.full((16,), i, jnp.int32)
        for c in range(D // 16):
            plsc.store_scatter(zb, [isp, c * 16 + lane], z16)
        return 0
    lax.fori_loop(0, 125, zfill, 0)
    for k in range(NROW // 125):
        pltpu.sync_copy(zb, acc_sp.at[pl.ds(sid * NROW + k * 125, 125)])
    plsc.subcore_barrier()

    def blk_body(b, _):
        bsp16 = jnp.full((16,), b, jnp.int32)
        cp = pltpu.async_copy(xl_hbm.at[srcv.at[b]], xrows, sem)

        def nb(k, _):
            kl = k * 16 + lane
            sv = plsc.load_gather(srcv, [bsp16, kl])
            nv = plsc.load_gather(normv, [sv])
            plsc.store_scatter(narr, [kl], nv)
            return 0
        lax.fori_loop(0, BLK // 16, nb, 0)
        cp.wait()

        def eb(e, _):
            esp = jnp.full((16,), e, jnp.int32)
            nsp = plsc.load_gather(narr, [esp])
            bv = plsc.load_gather(bidxv, [bsp16, esp])
            for c in range(D // 16):
                col = c * 16 + lane
                cv = plsc.load_gather(ctabv, [bv, col])
                xv = plsc.load_gather(xrows, [esp, col])
                m = nsp * jnp.maximum(xv + cv, 0.0)
                plsc.store_scatter(xrows, [esp, col], m)
            return 0
        lax.fori_loop(0, BLK, eb, 0)

        pltpu.sync_copy(xrows, acc_sp.at[dstv.at[b]], add=True)
        return 0
    lax.fori_loop(0, NBLK, blk_body, 0)

    plsc.subcore_barrier()
    pltpu.sync_copy(acc_sp.at[pl.ds(sid * NROW, NROW)],
                    out_hbm.at[cid, pl.ds(sid * NROW, NROW)])


_edge_call = pl.kernel(
    _edge_body,
    out_type=jax.ShapeDtypeStruct((NC, N, D), jnp.float32),
    mesh=plsc.VectorSubcoreMesh(core_axis_name="c", subcore_axis_name="s"),
    scratch_types=[
        pltpu.VMEM((NBLK, BLK), jnp.int32),
        pltpu.VMEM((NBLK, BLK), jnp.int32),
        pltpu.VMEM((NBLK, BLK), jnp.int32),
        pltpu.VMEM((N,), jnp.float32),
        pltpu.VMEM((VB ** FB, D), jnp.float32),
        pltpu.VMEM((BLK, D), jnp.float32),
        pltpu.VMEM((BLK,), jnp.float32),
        pltpu.VMEM((125, D), jnp.float32),
        pltpu.VMEM_SHARED((N, D), jnp.float32),
        pltpu.SemaphoreType.DMA,
    ],
)


def _prep_body(x_ref, deg2_ref, at_ref, w0_ref, b0_ref, bt_ref,
               ex0_ref, ex1_ref, ex2_ref,
               norm_ref, invd_ref, xl0_ref, ctabs_ref, bidx_ref):
    degs = deg2_ref[0] + deg2_ref[1] + 1.0
    norm_ref[...] = lax.rsqrt(degs)
    invd_ref[...] = 1.0 / degs

    iota = lax.broadcasted_iota(jnp.int32, (1, VA), 1)
    h = jnp.zeros((N, D), jnp.float32)
    for f in range(FA):
        oh = (x_ref[:, f][:, None] == iota).astype(jnp.float32)
        h = h + lax.dot(oh, at_ref[f], preferred_element_type=jnp.float32)
    xl0_ref[...] = lax.dot_general(
        h, w0_ref[...], (((1,), (1,)), ((), ())),
        preferred_element_type=jnp.float32) + b0_ref[...][None, :]

    for i in range(L):
        ct = (bt_ref[i, 0][:, None, None, :]
              + bt_ref[i, 1][None, :, None, :]
              + bt_ref[i, 2][None, None, :, :]).reshape(VB ** FB, D)
        ctabs_ref[i] = ct

    bidx_ref[...] = ex0_ref[...] * (VB * VB) + ex1_ref[...] * VB + ex2_ref[...]


_prep_call = pl.pallas_call(
    _prep_body,
    out_shape=[
        jax.ShapeDtypeStruct((N,), jnp.float32),
        jax.ShapeDtypeStruct((N,), jnp.float32),
        jax.ShapeDtypeStruct((N, D), jnp.float32),
        jax.ShapeDtypeStruct((L, VB ** FB, D), jnp.float32),
        jax.ShapeDtypeStruct((E,), jnp.int32),
    ],
)


def _upd_body(acc2_ref, xl_ref, norm_ref, invd_ref, root_ref, g_ref, b_ref,
              wn_ref, bn_ref, out_ref):
    acc = acc2_ref[0] + acc2_ref[1]
    h = (norm_ref[...][:, None] * acc
         + jnp.maximum(xl_ref[...] + root_ref[...][None, :], 0.0)
         * invd_ref[...][:, None])
    scale = g_ref[...] * (1.0 / jnp.sqrt(1.0 + EPS))
    h = h * scale[None, :] + b_ref[...][None, :]
    h = jnp.maximum(h, 0.0)
    out_ref[...] = lax.dot_general(
        h, wn_ref[...], (((1,), (1,)), ((), ())),
        preferred_element_type=jnp.float32) + bn_ref[...][None, :]


_upd_call = pl.pallas_call(
    _upd_body,
    out_shape=jax.ShapeDtypeStruct((N, D), jnp.float32),
)


def _fin_body(acc2_ref, xl_ref, norm_ref, invd_ref, root_ref, wo_ref, bo_ref,
              out_ref):
    acc = acc2_ref[0] + acc2_ref[1]
    h = (norm_ref[...][:, None] * acc
         + jnp.maximum(xl_ref[...] + root_ref[...][None, :], 0.0)
         * invd_ref[...][:, None])
    hg = jnp.sum(h, axis=0, keepdims=True) * (1.0 / N)
    out_ref[...] = lax.dot_general(
        hg, wo_ref[...], (((1,), (1,)), ((), ())),
        preferred_element_type=jnp.float32) + bo_ref[...][None, :]


_fin_call = pl.pallas_call(
    _fin_body,
    out_shape=jax.ShapeDtypeStruct((1, OUT), jnp.float32),
)


@jax.jit
def kernel(x, edge_index, ex, atom_tables, bond_tables, Ws, bs, roots,
           bn_gamma, bn_beta, W_out, b_out):
    src = edge_index[0].astype(jnp.int32)
    dst = edge_index[1].astype(jnp.int32)
    src2 = src.reshape(NT, EC)
    src3 = src.reshape(NT, NBLK, BLK)
    dst3 = dst.reshape(NT, NBLK, BLK)
    ex = ex.astype(jnp.int32)
    ex0, ex1, ex2 = ex[:, 0], ex[:, 1], ex[:, 2]

    deg2 = _deg_call(src2)
    norm, invd, xl, ctabs, bidx = _prep_call(
        x.astype(jnp.int32), deg2, atom_tables, Ws[0], bs[0], bond_tables,
        ex0, ex1, ex2)
    bidx3 = bidx.reshape(NT, NBLK, BLK)

    for i in range(L):
        acc2 = _edge_call(xl, src3, dst3, bidx3, norm, ctabs[i])
        if i < L - 1:
            xl = _upd_call(acc2, xl, norm, invd, roots[i], bn_gamma[i],
                           bn_beta[i], Ws[i + 1], bs[i + 1])
        else:
            out = _fin_call(acc2, xl, norm, invd, roots[L - 1], W_out, b_out)
    return out


# speedup vs baseline: 3.6319x; 3.6319x over previous
import functools

import jax
import jax.numpy as jnp
from jax import lax
from jax.experimental import pallas as pl
from jax.experimental.pallas import tpu as pltpu
from jax.experimental.pallas import tpu_sc as plsc

N = 10000
E = 320000
D = 128
L = 4
FA = 9
FB = 3
VA = 100
VB = 5
OUT = 128
EPS = 1e-5

NC = 2
NS = 16
NT = NC * NS
EC = E // NT
BLK = 80
NBLK = EC // BLK
NROW = N // NS


def _deg_body(src2_hbm, deg2_hbm, srcf_v, ones_v, zb_v, deg_sp, sem):
    del sem
    cid = lax.axis_index("c")
    sid = lax.axis_index("s")
    chunk = cid * NS + sid

    def fill(i, _):
        ones_v[pl.ds(i * 16, 16)] = jnp.full((16,), 1.0, jnp.float32)
        return 0
    lax.fori_loop(0, EC // 16, fill, 0)

    def zfill(i, _):
        zb_v[pl.ds(i * 16, 16)] = jnp.zeros((16,), jnp.float32)
        return 0
    lax.fori_loop(0, 2000 // 16, zfill, 0)

    @pl.when(sid < 5)
    def _():
        pltpu.sync_copy(zb_v, deg_sp.at[pl.ds(sid * 2000, 2000)])

    pltpu.sync_copy(src2_hbm.at[chunk, 0], srcf_v)
    plsc.subcore_barrier()
    pltpu.sync_copy(ones_v, deg_sp.at[srcf_v], add=True)
    plsc.subcore_barrier()

    @pl.when(sid == 0)
    def _():
        pltpu.sync_copy(deg_sp, deg2_hbm.at[cid, 0])


_SC_PARAMS = pltpu.CompilerParams(needs_layout_passes=False)

_deg_call = pl.kernel(
    _deg_body,
    out_type=jax.ShapeDtypeStruct((NC, 1, N), jnp.float32),
    compiler_params=_SC_PARAMS,
    mesh=plsc.VectorSubcoreMesh(core_axis_name="c", subcore_axis_name="s"),
    scratch_types=[
        pltpu.VMEM((EC,), jnp.int32),
        pltpu.VMEM((EC,), jnp.float32),
        pltpu.VMEM((2000,), jnp.float32),
        pltpu.VMEM_SHARED((N,), jnp.float32),
        pltpu.SemaphoreType.DMA,
    ],
)


def _edge_body(xl_hbm, pk2_hbm, dst2_hbm, norm_hbm, ctab_hbm, zeros_hbm,
               out_hbm,
               pkv, dstv, normv, xrows, crows, narr,
               ctab_sp, acc_sp, sem, sem2):
    cid = lax.axis_index("c")
    sid = lax.axis_index("s")
    chunk = cid * NS + sid

    pltpu.sync_copy(pk2_hbm.at[chunk, 0], pkv)
    pltpu.sync_copy(dst2_hbm.at[chunk, 0], dstv)
    pltpu.sync_copy(norm_hbm, normv)

    @pl.when(sid == 0)
    def _():
        pltpu.sync_copy(ctab_hbm, ctab_sp)

    @pl.when(sid % 8 == 0)
    def _():
        start = (sid // 8) * (N // 2)
        pltpu.sync_copy(zeros_hbm.at[pl.ds(start, N // 2)],
                        acc_sp.at[pl.ds(start, N // 2)])
    plsc.subcore_barrier()

    def grp_body(g, _):
        w = pkv[pl.ds(g * 16, 16)]
        sv = w & 0x3FFF
        bv = w >> 14
        dv = dstv[pl.ds(g * 16, 16)]
        cp1 = pltpu.async_copy(xl_hbm.at[sv], xrows, sem)
        cp2 = pltpu.async_copy(ctab_sp.at[bv], crows, sem2)
        narr[...] = plsc.load_gather(normv, [sv])
        cp1.wait()
        cp2.wait()

        def eb(e, _):
            esp = jnp.full((16,), e, jnp.int32)
            nsp = plsc.load_gather(narr, [esp])
            for c in range(D // 16):
                sl = pl.ds(c * 16, 16)
                v = xrows[e, sl] + crows[e, sl]
                xrows[e, sl] = nsp * jnp.maximum(v, 0.0)
            return 0
        lax.fori_loop(0, 16, eb, 0)

        pltpu.sync_copy(xrows, acc_sp.at[dv], add=True)
        return 0
    lax.fori_loop(0, EC // 16, grp_body, 0)

    plsc.subcore_barrier()

    @pl.when(sid % 8 == 0)
    def _():
        start = (sid // 8) * (N // 2)
        pltpu.sync_copy(acc_sp.at[pl.ds(start, N // 2)],
                        out_hbm.at[cid, pl.ds(start, N // 2)])


_edge_call = pl.kernel(
    _edge_body,
    out_type=jax.ShapeDtypeStruct((NC, N, D), jnp.float32),
    compiler_params=_SC_PARAMS,
    mesh=plsc.VectorSubcoreMesh(core_axis_name="c", subcore_axis_name="s"),
    scratch_types=[
        pltpu.VMEM((EC,), jnp.int32),
        pltpu.VMEM((EC,), jnp.int32),
        pltpu.VMEM((N,), jnp.float32),
        pltpu.VMEM((16, D), jnp.float32),
        pltpu.VMEM((16, D), jnp.float32),
        pltpu.VMEM((16,), jnp.float32),
        pltpu.VMEM_SHARED((VB ** FB, D), jnp.float32),
        pltpu.VMEM_SHARED((N, D), jnp.float32),
        pltpu.SemaphoreType.DMA,
        pltpu.SemaphoreType.DMA,
    ],
)


def _prep_body(x_ref, deg2_ref, at_ref, w0_ref, b0_ref, bt_ref,
               ex0_ref, ex1_ref, ex2_ref, src_ref,
               norm_ref, invd_ref, xl0_ref, ctabs_ref, pk_ref):
    degs = deg2_ref[0, 0] + deg2_ref[1, 0] + 1.0
    norm_ref[...] = lax.rsqrt(degs)
    invd_ref[...] = 1.0 / degs

    iota = lax.broadcasted_iota(jnp.int32, (1, VA), 1)
    h = jnp.zeros((N, D), jnp.float32)
    for f in range(FA):
        oh = (x_ref[:, f][:, None] == iota).astype(jnp.float32)
        h = h + lax.dot(oh, at_ref[f], preferred_element_type=jnp.float32)
    xl0_ref[...] = lax.dot_general(
        h, w0_ref[...], (((1,), (1,)), ((), ())),
        preferred_element_type=jnp.float32) + b0_ref[...][None, :]

    for i in range(L):
        ct = (bt_ref[i, 0][:, None, None, :]
              + bt_ref[i, 1][None, :, None, :]
              + bt_ref[i, 2][None, None, :, :]).reshape(VB ** FB, D)
        ctabs_ref[i] = ct

    bidx = ex0_ref[...] * (VB * VB) + ex1_ref[...] * VB + ex2_ref[...]
    pk_ref[...] = src_ref[...] + bidx * 16384


_prep_call = pl.pallas_call(
    _prep_body,
    out_shape=[
        jax.ShapeDtypeStruct((N,), jnp.float32),
        jax.ShapeDtypeStruct((N,), jnp.float32),
        jax.ShapeDtypeStruct((N, D), jnp.float32),
        jax.ShapeDtypeStruct((L, VB ** FB, D), jnp.float32),
        jax.ShapeDtypeStruct((E,), jnp.int32),
    ],
)


def _upd_body(acc2_ref, xl_ref, norm_ref, invd_ref, root_ref, g_ref, b_ref,
              wn_ref, bn_ref, out_ref):
    acc = acc2_ref[0] + acc2_ref[1]
    h = (norm_ref[...][:, None] * acc
         + jnp.maximum(xl_ref[...] + root_ref[...][None, :], 0.0)
         * invd_ref[...][:, None])
    scale = g_ref[...] * (1.0 / jnp.sqrt(1.0 + EPS))
    h = h * scale[None, :] + b_ref[...][None, :]
    h = jnp.maximum(h, 0.0)
    out_ref[...] = lax.dot_general(
        h, wn_ref[...], (((1,), (1,)), ((), ())),
        preferred_element_type=jnp.float32) + bn_ref[...][None, :]


_upd_call = pl.pallas_call(
    _upd_body,
    out_shape=jax.ShapeDtypeStruct((N, D), jnp.float32),
)


def _fin_body(acc2_ref, xl_ref, norm_ref, invd_ref, root_ref, wo_ref, bo_ref,
              out_ref):
    acc = acc2_ref[0] + acc2_ref[1]
    h = (norm_ref[...][:, None] * acc
         + jnp.maximum(xl_ref[...] + root_ref[...][None, :], 0.0)
         * invd_ref[...][:, None])
    hg = jnp.sum(h, axis=0, keepdims=True) * (1.0 / N)
    out_ref[...] = lax.dot_general(
        hg, wo_ref[...], (((1,), (1,)), ((), ())),
        preferred_element_type=jnp.float32) + bo_ref[...][None, :]


_fin_call = pl.pallas_call(
    _fin_body,
    out_shape=jax.ShapeDtypeStruct((1, OUT), jnp.float32),
)


@jax.jit
def kernel(x, edge_index, ex, atom_tables, bond_tables, Ws, bs, roots,
           bn_gamma, bn_beta, W_out, b_out):
    src = edge_index[0].astype(jnp.int32)
    dst = edge_index[1].astype(jnp.int32)
    src2 = src.reshape(NT, 1, EC)
    dst2 = dst.reshape(NT, 1, EC)
    ex = ex.astype(jnp.int32)
    ex0, ex1, ex2 = ex[:, 0], ex[:, 1], ex[:, 2]

    deg2 = _deg_call(src2)
    norm, invd, xl, ctabs, pk = _prep_call(
        x.astype(jnp.int32), deg2, atom_tables, Ws[0], bs[0], bond_tables,
        ex0, ex1, ex2, src)
    pk2 = pk.reshape(NT, 1, EC)
    zeros = jnp.zeros((N, D), jnp.float32)

    for i in range(L):
        acc2 = _edge_call(xl, pk2, dst2, norm, ctabs[i], zeros)
        if i < L - 1:
            xl = _upd_call(acc2, xl, norm, invd, roots[i], bn_gamma[i],
                           bn_beta[i], Ws[i + 1], bs[i + 1])
        else:
            out = _fin_call(acc2, xl, norm, invd, roots[L - 1], W_out, b_out)
    return out
